# baseline (speedup 1.0000x reference)
import functools

import jax
import jax.numpy as jnp
from jax import lax
from jax.experimental import pallas as pl
from jax.experimental.pallas import tpu as pltpu
from jax.experimental.pallas import tpu_sc as plsc

N = 10000
E = 320000
D = 128

NW = 32
BLK = 128
K = 3
NSB = 27
NB = K * NSB
EPAD = NW * NB * BLK
NPAD = 10240
RPT = NPAD // 16

_mesh = plsc.VectorSubcoreMesh(core_axis_name="c", subcore_axis_name="s")


@functools.partial(
    pl.kernel,
    out_type=jax.ShapeDtypeStruct((2, NPAD, 16), jnp.float32),
    mesh=_mesh,
    compiler_params=pltpu.CompilerParams(use_tc_tiling_on_sc=False),
    scratch_types=[
        pltpu.VMEM((NB, BLK), jnp.int32),
        pltpu.VMEM((BLK, 16), jnp.float32),
        pltpu.VMEM((BLK, 16), jnp.float32),
        pltpu.VMEM_SHARED((NPAD, 16), jnp.float32),
    ],
)
def _sc_deg(dst_hbm, ones_hbm, zeros_hbm, out_hbm, idx_v, ones_v, zeros_v, acc):
    c = lax.axis_index("c")
    s = lax.axis_index("s")
    wid = s * 2 + c
    pltpu.sync_copy(dst_hbm.at[wid], idx_v)
    pltpu.sync_copy(ones_hbm, ones_v)
    pltpu.sync_copy(zeros_hbm, zeros_v)
    for b in range(RPT // BLK):
        pltpu.sync_copy(zeros_v, acc.at[pl.ds(s * RPT + b * BLK, BLK)])
    plsc.subcore_barrier()

    def body(j, carry):
        pltpu.sync_copy(ones_v, acc.at[idx_v.at[j]], add=True)
        return carry

    lax.fori_loop(0, NB, body, 0)
    plsc.subcore_barrier()
    pltpu.sync_copy(acc.at[pl.ds(s * RPT, RPT)], out_hbm.at[c, pl.ds(s * RPT, RPT)])


DH = D // 2


@functools.partial(
    pl.kernel,
    out_type=jax.ShapeDtypeStruct((2, 2, NPAD, DH), jnp.float32),
    mesh=_mesh,
    compiler_params=pltpu.CompilerParams(use_tc_tiling_on_sc=False),
    scratch_types=[
        pltpu.VMEM((NB, BLK), jnp.int32),
        pltpu.VMEM((NB, BLK), jnp.int32),
        pltpu.VMEM((2, BLK, DH), jnp.float32),
        pltpu.VMEM((BLK, DH), jnp.float32),
        pltpu.VMEM_SHARED((NPAD, DH), jnp.float32),
        pltpu.SemaphoreType.DMA,
        pltpu.SemaphoreType.DMA,
    ],
)
def _sc_scatter(gl_hbm, gr_hbm, src_hbm, dst_hbm, zeros_hbm, out_hbm,
                src_v, dst_v, bufs, zeros_v, acc, sem_ga, sem_gb):
    c = lax.axis_index("c")
    s = lax.axis_index("s")
    wid = s * 2 + c
    pltpu.sync_copy(src_hbm.at[wid], src_v)
    pltpu.sync_copy(dst_hbm.at[wid], dst_v)
    pltpu.sync_copy(zeros_hbm, zeros_v)
    for half in range(2):
        g_hbm = gl_hbm if half == 0 else gr_hbm

        def fire(j, st):
            pltpu.async_copy(g_hbm.at[src_v.at[j]], bufs.at[st],
                             sem_ga if st == 0 else sem_gb)

        def drain(j, st):
            pltpu.make_async_copy(g_hbm.at[src_v.at[j]], bufs.at[st],
                                  sem_ga if st == 0 else sem_gb).wait()

        def scat(j, st):
            pltpu.sync_copy(bufs.at[st], acc.at[dst_v.at[j]], add=True)

        for b in range(RPT // BLK):
            pltpu.sync_copy(zeros_v, acc.at[pl.ds(s * RPT + b * BLK, BLK)])
        plsc.subcore_barrier()

        fire(0, 0)

        def body(i, carry):
            j0 = 2 * i
            j1 = 2 * i + 1
            drain(j0, 0)
            fire(j1, 1)
            scat(j0, 0)
            drain(j1, 1)
            fire(j1 + 1, 0)
            scat(j1, 1)
            return carry

        lax.fori_loop(0, (NB - 1) // 2, body, 0)
        drain(NB - 1, 0)
        scat(NB - 1, 0)
        plsc.subcore_barrier()
        pltpu.sync_copy(acc.at[pl.ds(s * RPT, RPT)],
                        out_hbm.at[c, half, pl.ds(s * RPT, RPT)])


_TC_BLK = 1024
_TC_GRID = NPAD // _TC_BLK


def _rows_spec(width=D):
    return pl.BlockSpec((_TC_BLK, width), lambda i: (i, 0))


def _full_spec(shape):
    return pl.BlockSpec(shape, lambda i: (0, 0))


def _pdeg_spec(ci):
    return pl.BlockSpec((1, _TC_BLK, 16), lambda i, ci=ci: (ci, i, 0))


def _u_spec(ci, hi):
    return pl.BlockSpec((1, 1, _TC_BLK, DH), lambda i, ci=ci, hi=hi: (ci, hi, i, 0))


def _dinv(pa_ref, pb_ref):
    deg = 1.0 + pa_ref[0, :, :1] + pb_ref[0, :, :1]
    return lax.rsqrt(deg)


def _usum(u0l, u1l, u0r, u1r):
    return jnp.concatenate([u0l[0, 0] + u1l[0, 0], u0r[0, 0] + u1r[0, 0]],
                           axis=1)


def _tc1_body(x_ref, w_ref, pa_ref, pb_ref, h_ref, gl_ref, gr_ref):
    h = jnp.dot(x_ref[...], w_ref[...], preferred_element_type=jnp.float32)
    dinv = _dinv(pa_ref, pb_ref)
    h_ref[...] = h
    g = h * dinv
    gl_ref[...] = g[:, :DH]
    gr_ref[...] = g[:, DH:]


def _tc2_body(u0l, u1l, u0r, u1r, h1_ref, pa_ref, pb_ref, b1_ref, w2_ref,
              h2_ref, g2l_ref, g2r_ref):
    dinv = _dinv(pa_ref, pb_ref)
    a = dinv * _usum(u0l, u1l, u0r, u1r) + (dinv * dinv) * h1_ref[...] + b1_ref[...]
    o = jnp.maximum(a, 0.0)
    h2 = jnp.dot(o, w2_ref[...], preferred_element_type=jnp.float32)
    h2_ref[...] = h2
    g2 = h2 * dinv
    g2l_ref[...] = g2[:, :DH]
    g2r_ref[...] = g2[:, DH:]


def _tc3_body(u0l, u1l, u0r, u1r, h2_ref, pa_ref, pb_ref, b2_ref, out_ref):
    dinv = _dinv(pa_ref, pb_ref)
    out_ref[...] = (dinv * _usum(u0l, u1l, u0r, u1r)
                    + (dinv * dinv) * h2_ref[...] + b2_ref[...])


_half_out = [jax.ShapeDtypeStruct((NPAD, DH), jnp.float32)] * 2
_u_specs = [_u_spec(0, 0), _u_spec(1, 0), _u_spec(0, 1), _u_spec(1, 1)]

_tc1 = pl.pallas_call(
    _tc1_body,
    grid=(_TC_GRID,),
    in_specs=[_rows_spec(), _full_spec((D, D)), _pdeg_spec(0), _pdeg_spec(1)],
    out_specs=[_rows_spec(), _rows_spec(DH), _rows_spec(DH)],
    out_shape=[jax.ShapeDtypeStruct((NPAD, D), jnp.float32)] + _half_out,
)

_tc2 = pl.pallas_call(
    _tc2_body,
    grid=(_TC_GRID,),
    in_specs=_u_specs + [_rows_spec(), _pdeg_spec(0), _pdeg_spec(1),
                         _full_spec((1, D)), _full_spec((D, D))],
    out_specs=[_rows_spec(), _rows_spec(DH), _rows_spec(DH)],
    out_shape=[jax.ShapeDtypeStruct((NPAD, D), jnp.float32)] + _half_out,
)

_tc3 = pl.pallas_call(
    _tc3_body,
    grid=(_TC_GRID,),
    in_specs=_u_specs + [_rows_spec(), _pdeg_spec(0), _pdeg_spec(1),
                         _full_spec((1, D))],
    out_specs=_rows_spec(),
    out_shape=jax.ShapeDtypeStruct((NPAD, D), jnp.float32),
)


def kernel(x, edge_index, W1, b1, W2, b2):
    ei = edge_index.astype(jnp.int32)
    pad = jnp.full((EPAD - E,), N, dtype=jnp.int32)
    src = jnp.concatenate([ei[0], pad]).reshape(NW, NB, BLK)
    dst = jnp.concatenate([ei[1], pad]).reshape(NW, NB, BLK)
    xp = jnp.pad(x, ((0, NPAD - N), (0, 0)))
    zeros_h = jnp.zeros((BLK, DH), jnp.float32)
    ones16 = jnp.ones((BLK, 16), jnp.float32)
    zeros16 = jnp.zeros((BLK, 16), jnp.float32)

    pdeg = _sc_deg(dst, ones16, zeros16)

    h1, g1l, g1r = _tc1(xp, W1, pdeg, pdeg)
    u1 = _sc_scatter(g1l, g1r, src, dst, zeros_h)
    h2, g2l, g2r = _tc2(u1, u1, u1, u1, h1, pdeg, pdeg,
                        b1.reshape(1, D), W2)
    u2 = _sc_scatter(g2l, g2r, src, dst, zeros_h)
    out = _tc3(u2, u2, u2, u2, h2, pdeg, pdeg, b2.reshape(1, D))
    return out[:N]


# speedup vs baseline: 1.0105x; 1.0105x over previous
import functools

import jax
import jax.numpy as jnp
from jax import lax
from jax.experimental import pallas as pl
from jax.experimental.pallas import tpu as pltpu
from jax.experimental.pallas import tpu_sc as plsc

N = 10000
E = 320000
D = 128

NW = 32
BLK = 128
K = 3
NSB = 27
NB = K * NSB
EPAD = NW * NB * BLK
NPAD = 10240
RPT = NPAD // 16

_mesh = plsc.VectorSubcoreMesh(core_axis_name="c", subcore_axis_name="s")


@functools.partial(
    pl.kernel,
    out_type=jax.ShapeDtypeStruct((2, NPAD, 16), jnp.float32),
    mesh=_mesh,
    compiler_params=pltpu.CompilerParams(use_tc_tiling_on_sc=False),
    scratch_types=[
        pltpu.VMEM((NB, BLK), jnp.int32),
        pltpu.VMEM((BLK, 16), jnp.float32),
        pltpu.VMEM((BLK, 16), jnp.float32),
        pltpu.VMEM_SHARED((NPAD, 16), jnp.float32),
    ],
)
def _sc_deg(dst_hbm, ones_hbm, zeros_hbm, out_hbm, idx_v, ones_v, zeros_v, acc):
    c = lax.axis_index("c")
    s = lax.axis_index("s")
    wid = s * 2 + c
    pltpu.sync_copy(dst_hbm.at[wid], idx_v)
    pltpu.sync_copy(ones_hbm, ones_v)
    pltpu.sync_copy(zeros_hbm, zeros_v)
    for b in range(RPT // BLK):
        pltpu.sync_copy(zeros_v, acc.at[pl.ds(s * RPT + b * BLK, BLK)])
    plsc.subcore_barrier()

    def body(j, carry):
        pltpu.sync_copy(ones_v, acc.at[idx_v.at[j]], add=True)
        return carry

    lax.fori_loop(0, NB, body, 0)
    plsc.subcore_barrier()
    pltpu.sync_copy(acc.at[pl.ds(s * RPT, RPT)], out_hbm.at[c, pl.ds(s * RPT, RPT)])


DH = D // 2


@functools.partial(
    pl.kernel,
    out_type=jax.ShapeDtypeStruct((2, 2, NPAD, DH), jnp.float32),
    mesh=_mesh,
    compiler_params=pltpu.CompilerParams(use_tc_tiling_on_sc=False),
    scratch_types=[
        pltpu.VMEM((NB, BLK), jnp.int32),
        pltpu.VMEM((NB, BLK), jnp.int32),
        pltpu.VMEM((BLK, DH), jnp.float32),
        pltpu.VMEM((BLK, DH), jnp.float32),
        pltpu.VMEM((BLK, DH), jnp.float32),
        pltpu.VMEM_SHARED((NPAD, DH), jnp.float32),
        pltpu.SemaphoreType.DMA,
        pltpu.SemaphoreType.DMA,
    ],
)
def _sc_scatter(gl_hbm, gr_hbm, src_hbm, dst_hbm, zeros_hbm, out_hbm,
                src_v, dst_v, rows_a, rows_b, zeros_v, acc, sem_ga, sem_gb):
    c = lax.axis_index("c")
    s = lax.axis_index("s")
    wid = s * 2 + c
    pltpu.sync_copy(src_hbm.at[wid], src_v)
    pltpu.sync_copy(dst_hbm.at[wid], dst_v)
    pltpu.sync_copy(zeros_hbm, zeros_v)
    for half in range(2):
        g_hbm = gl_hbm if half == 0 else gr_hbm

        def fire(j, st):
            pltpu.async_copy(g_hbm.at[src_v.at[j]],
                             rows_a if st == 0 else rows_b,
                             sem_ga if st == 0 else sem_gb)

        def drain(j, st):
            pltpu.make_async_copy(g_hbm.at[src_v.at[j]],
                                  rows_a if st == 0 else rows_b,
                                  sem_ga if st == 0 else sem_gb).wait()

        def scat(j, st):
            pltpu.sync_copy(rows_a if st == 0 else rows_b,
                            acc.at[dst_v.at[j]], add=True)

        for b in range(RPT // BLK):
            pltpu.sync_copy(zeros_v, acc.at[pl.ds(s * RPT + b * BLK, BLK)])
        plsc.subcore_barrier()

        fire(0, 0)

        def body(i, carry):
            j0 = 2 * i
            j1 = 2 * i + 1
            drain(j0, 0)
            fire(j1, 1)
            scat(j0, 0)
            drain(j1, 1)
            fire(j1 + 1, 0)
            scat(j1, 1)
            return carry

        lax.fori_loop(0, (NB - 1) // 2, body, 0)
        drain(NB - 1, 0)
        scat(NB - 1, 0)
        plsc.subcore_barrier()
        pltpu.sync_copy(acc.at[pl.ds(s * RPT, RPT)],
                        out_hbm.at[c, half, pl.ds(s * RPT, RPT)])


_TC_BLK = 1024
_TC_GRID = NPAD // _TC_BLK


def _rows_spec(width=D):
    return pl.BlockSpec((_TC_BLK, width), lambda i: (i, 0))


def _full_spec(shape):
    return pl.BlockSpec(shape, lambda i: (0, 0))


def _pdeg_spec(ci):
    return pl.BlockSpec((1, _TC_BLK, 16), lambda i, ci=ci: (ci, i, 0))


def _u_spec(ci, hi):
    return pl.BlockSpec((1, 1, _TC_BLK, DH), lambda i, ci=ci, hi=hi: (ci, hi, i, 0))


def _dinv(pa_ref, pb_ref):
    deg = 1.0 + pa_ref[0, :, :1] + pb_ref[0, :, :1]
    return lax.rsqrt(deg)


def _usum(u0l, u1l, u0r, u1r):
    return jnp.concatenate([u0l[0, 0] + u1l[0, 0], u0r[0, 0] + u1r[0, 0]],
                           axis=1)


def _tc1_body(x_ref, w_ref, pa_ref, pb_ref, h_ref, gl_ref, gr_ref):
    h = jnp.dot(x_ref[...], w_ref[...], preferred_element_type=jnp.float32)
    dinv = _dinv(pa_ref, pb_ref)
    h_ref[...] = h
    g = h * dinv
    gl_ref[...] = g[:, :DH]
    gr_ref[...] = g[:, DH:]


def _tc2_body(u0l, u1l, u0r, u1r, h1_ref, pa_ref, pb_ref, b1_ref, w2_ref,
              h2_ref, g2l_ref, g2r_ref):
    dinv = _dinv(pa_ref, pb_ref)
    a = dinv * _usum(u0l, u1l, u0r, u1r) + (dinv * dinv) * h1_ref[...] + b1_ref[...]
    o = jnp.maximum(a, 0.0)
    h2 = jnp.dot(o, w2_ref[...], preferred_element_type=jnp.float32)
    h2_ref[...] = h2
    g2 = h2 * dinv
    g2l_ref[...] = g2[:, :DH]
    g2r_ref[...] = g2[:, DH:]


def _tc3_body(u0l, u1l, u0r, u1r, h2_ref, pa_ref, pb_ref, b2_ref, out_ref):
    dinv = _dinv(pa_ref, pb_ref)
    out_ref[...] = (dinv * _usum(u0l, u1l, u0r, u1r)
                    + (dinv * dinv) * h2_ref[...] + b2_ref[...])


_half_out = [jax.ShapeDtypeStruct((NPAD, DH), jnp.float32)] * 2
_u_specs = [_u_spec(0, 0), _u_spec(1, 0), _u_spec(0, 1), _u_spec(1, 1)]

_tc1 = pl.pallas_call(
    _tc1_body,
    grid=(_TC_GRID,),
    in_specs=[_rows_spec(), _full_spec((D, D)), _pdeg_spec(0), _pdeg_spec(1)],
    out_specs=[_rows_spec(), _rows_spec(DH), _rows_spec(DH)],
    out_shape=[jax.ShapeDtypeStruct((NPAD, D), jnp.float32)] + _half_out,
)

_tc2 = pl.pallas_call(
    _tc2_body,
    grid=(_TC_GRID,),
    in_specs=_u_specs + [_rows_spec(), _pdeg_spec(0), _pdeg_spec(1),
                         _full_spec((1, D)), _full_spec((D, D))],
    out_specs=[_rows_spec(), _rows_spec(DH), _rows_spec(DH)],
    out_shape=[jax.ShapeDtypeStruct((NPAD, D), jnp.float32)] + _half_out,
)

_tc3 = pl.pallas_call(
    _tc3_body,
    grid=(_TC_GRID,),
    in_specs=_u_specs + [_rows_spec(), _pdeg_spec(0), _pdeg_spec(1),
                         _full_spec((1, D))],
    out_specs=_rows_spec(),
    out_shape=jax.ShapeDtypeStruct((NPAD, D), jnp.float32),
)


def kernel(x, edge_index, W1, b1, W2, b2):
    ei = edge_index.astype(jnp.int32)
    pad = jnp.full((EPAD - E,), N, dtype=jnp.int32)
    src = jnp.concatenate([ei[0], pad]).reshape(NW, NB, BLK)
    dst = jnp.concatenate([ei[1], pad]).reshape(NW, NB, BLK)
    xp = jnp.pad(x, ((0, NPAD - N), (0, 0)))
    zeros_h = jnp.zeros((BLK, DH), jnp.float32)
    ones16 = jnp.ones((BLK, 16), jnp.float32)
    zeros16 = jnp.zeros((BLK, 16), jnp.float32)

    pdeg = _sc_deg(dst, ones16, zeros16)

    h1, g1l, g1r = _tc1(xp, W1, pdeg, pdeg)
    u1 = _sc_scatter(g1l, g1r, src, dst, zeros_h)
    h2, g2l, g2r = _tc2(u1, u1, u1, u1, h1, pdeg, pdeg,
                        b1.reshape(1, D), W2)
    u2 = _sc_scatter(g2l, g2r, src, dst, zeros_h)
    out = _tc3(u2, u2, u2, u2, h2, pdeg, pdeg, b2.reshape(1, D))
    return out[:N]


# speedup vs baseline: 2.6293x; 2.6020x over previous
import functools

import jax
import jax.numpy as jnp
from jax import lax
from jax.experimental import pallas as pl
from jax.experimental.pallas import tpu as pltpu
from jax.experimental.pallas import tpu_sc as plsc

N = 10000
E = 320000
D = 128

NW = 32
BLK = 128
K = 3
NSB = 27
NB = K * NSB
EPAD = NW * NB * BLK
NPAD = 10240
RPT = NPAD // 16

_mesh = plsc.VectorSubcoreMesh(core_axis_name="c", subcore_axis_name="s")


@functools.partial(
    pl.kernel,
    out_type=jax.ShapeDtypeStruct((2, NPAD, 16), jnp.float32),
    mesh=_mesh,
    compiler_params=pltpu.CompilerParams(use_tc_tiling_on_sc=False),
    scratch_types=[
        pltpu.VMEM((NB, BLK), jnp.int32),
        pltpu.VMEM((BLK, 16), jnp.float32),
        pltpu.VMEM((BLK, 16), jnp.float32),
        pltpu.VMEM_SHARED((NPAD, 16), jnp.float32),
    ],
)
def _sc_deg(dst_hbm, ones_hbm, zeros_hbm, out_hbm, idx_v, ones_v, zeros_v, acc):
    c = lax.axis_index("c")
    s = lax.axis_index("s")
    wid = s * 2 + c
    pltpu.sync_copy(dst_hbm.at[wid], idx_v)
    pltpu.sync_copy(ones_hbm, ones_v)
    pltpu.sync_copy(zeros_hbm, zeros_v)
    for b in range(RPT // BLK):
        pltpu.sync_copy(zeros_v, acc.at[pl.ds(s * RPT + b * BLK, BLK)])
    plsc.subcore_barrier()

    def body(j, carry):
        pltpu.sync_copy(ones_v, acc.at[idx_v.at[j]], add=True)
        return carry

    lax.fori_loop(0, NB, body, 0)
    plsc.subcore_barrier()
    pltpu.sync_copy(acc.at[pl.ds(s * RPT, RPT)], out_hbm.at[c, pl.ds(s * RPT, RPT)])


DH = D // 2


@functools.partial(
    pl.kernel,
    out_type=jax.ShapeDtypeStruct((2, 2, NPAD, DH), jnp.float32),
    mesh=_mesh,
    compiler_params=pltpu.CompilerParams(use_tc_tiling_on_sc=False),
    scratch_types=[
        pltpu.VMEM((NB, BLK), jnp.int32),
        pltpu.VMEM((NB, BLK), jnp.int32),
        pltpu.VMEM((BLK, DH), jnp.float32),
        pltpu.VMEM((BLK, DH), jnp.float32),
        pltpu.VMEM((BLK, DH), jnp.float32),
        pltpu.VMEM_SHARED((NPAD, DH), jnp.float32),
        pltpu.SemaphoreType.DMA,
        pltpu.SemaphoreType.DMA,
    ],
)
def _sc_scatter(gl_hbm, gr_hbm, src_hbm, dst_hbm, zeros_hbm, out_hbm,
                src_v, dst_v, rows_a, rows_b, zeros_v, acc, sem_ga, sem_gb):
    c = lax.axis_index("c")
    s = lax.axis_index("s")
    wid = s * 2 + c
    pltpu.sync_copy(src_hbm.at[wid], src_v)
    pltpu.sync_copy(dst_hbm.at[wid], dst_v)
    pltpu.sync_copy(zeros_hbm, zeros_v)
    for half in range(2):
        g_hbm = gl_hbm if half == 0 else gr_hbm

        def fire(j, st):
            pltpu.async_copy(g_hbm.at[src_v.at[j]],
                             rows_a if st == 0 else rows_b,
                             sem_ga if st == 0 else sem_gb)

        def drain(j, st):
            pltpu.make_async_copy(g_hbm.at[src_v.at[j]],
                                  rows_a if st == 0 else rows_b,
                                  sem_ga if st == 0 else sem_gb).wait()

        def scat(j, st):
            pltpu.sync_copy(rows_a if st == 0 else rows_b,
                            acc.at[dst_v.at[j]], add=True)

        for b in range(RPT // BLK):
            pltpu.sync_copy(zeros_v, acc.at[pl.ds(s * RPT + b * BLK, BLK)])
        plsc.subcore_barrier()

        fire(0, 0)

        def body(i, carry):
            j0 = 2 * i
            j1 = 2 * i + 1
            drain(j0, 0)
            fire(j1, 1)
            scat(j0, 0)
            drain(j1, 1)
            fire(j1 + 1, 0)
            scat(j1, 1)
            return carry

        lax.fori_loop(0, (NB - 1) // 2, body, 0)
        drain(NB - 1, 0)
        scat(NB - 1, 0)
        plsc.subcore_barrier()
        pltpu.sync_copy(acc.at[pl.ds(s * RPT, RPT)],
                        out_hbm.at[c, half, pl.ds(s * RPT, RPT)])


_TC_BLK = 1024
_TC_GRID = NPAD // _TC_BLK


def _rows_spec(width=D):
    return pl.BlockSpec((_TC_BLK, width), lambda i: (i, 0))


def _full_spec(shape):
    return pl.BlockSpec(shape, lambda i: (0, 0))


def _pdeg_spec(ci):
    return pl.BlockSpec((1, _TC_BLK, 16), lambda i, ci=ci: (ci, i, 0))


def _u_spec(ci, hi):
    return pl.BlockSpec((1, 1, _TC_BLK, DH), lambda i, ci=ci, hi=hi: (ci, hi, i, 0))


def _dinv(pa_ref, pb_ref):
    deg = 1.0 + pa_ref[0, :, :1] + pb_ref[0, :, :1]
    return lax.rsqrt(deg)


def _usum(u0l, u1l, u0r, u1r):
    return jnp.concatenate([u0l[0, 0] + u1l[0, 0], u0r[0, 0] + u1r[0, 0]],
                           axis=1)


def _tc1_body(x_ref, w_ref, pa_ref, pb_ref, h_ref, gl_ref, gr_ref):
    h = jnp.dot(x_ref[...], w_ref[...], preferred_element_type=jnp.float32)
    dinv = _dinv(pa_ref, pb_ref)
    h_ref[...] = h
    g = h * dinv
    gl_ref[...] = g[:, :DH]
    gr_ref[...] = g[:, DH:]


def _tc2_body(u0l, u1l, u0r, u1r, h1_ref, pa_ref, pb_ref, b1_ref, w2_ref,
              h2_ref, g2l_ref, g2r_ref):
    dinv = _dinv(pa_ref, pb_ref)
    a = dinv * _usum(u0l, u1l, u0r, u1r) + (dinv * dinv) * h1_ref[...] + b1_ref[...]
    o = jnp.maximum(a, 0.0)
    h2 = jnp.dot(o, w2_ref[...], preferred_element_type=jnp.float32)
    h2_ref[...] = h2
    g2 = h2 * dinv
    g2l_ref[...] = g2[:, :DH]
    g2r_ref[...] = g2[:, DH:]


def _tc3_body(u0l, u1l, u0r, u1r, h2_ref, pa_ref, pb_ref, b2_ref, out_ref):
    dinv = _dinv(pa_ref, pb_ref)
    out_ref[...] = (dinv * _usum(u0l, u1l, u0r, u1r)
                    + (dinv * dinv) * h2_ref[...] + b2_ref[...])


_half_out = [jax.ShapeDtypeStruct((NPAD, DH), jnp.float32)] * 2
_u_specs = [_u_spec(0, 0), _u_spec(1, 0), _u_spec(0, 1), _u_spec(1, 1)]

_tc1 = pl.pallas_call(
    _tc1_body,
    grid=(_TC_GRID,),
    in_specs=[_rows_spec(), _full_spec((D, D)), _pdeg_spec(0), _pdeg_spec(1)],
    out_specs=[_rows_spec(), _rows_spec(DH), _rows_spec(DH)],
    out_shape=[jax.ShapeDtypeStruct((NPAD, D), jnp.float32)] + _half_out,
)

_tc2 = pl.pallas_call(
    _tc2_body,
    grid=(_TC_GRID,),
    in_specs=_u_specs + [_rows_spec(), _pdeg_spec(0), _pdeg_spec(1),
                         _full_spec((1, D)), _full_spec((D, D))],
    out_specs=[_rows_spec(), _rows_spec(DH), _rows_spec(DH)],
    out_shape=[jax.ShapeDtypeStruct((NPAD, D), jnp.float32)] + _half_out,
)

_tc3 = pl.pallas_call(
    _tc3_body,
    grid=(_TC_GRID,),
    in_specs=_u_specs + [_rows_spec(), _pdeg_spec(0), _pdeg_spec(1),
                         _full_spec((1, D))],
    out_specs=_rows_spec(),
    out_shape=jax.ShapeDtypeStruct((NPAD, D), jnp.float32),
)


def kernel(x, edge_index, W1, b1, W2, b2):
    ei = edge_index.astype(jnp.int32)
    pad = N + jnp.arange(EPAD - E, dtype=jnp.int32) % (NPAD - N)
    src = jnp.concatenate([ei[0], pad]).reshape(NW, NB, BLK)
    dst = jnp.concatenate([ei[1], pad]).reshape(NW, NB, BLK)
    xp = jnp.pad(x, ((0, NPAD - N), (0, 0)))
    zeros_h = jnp.zeros((BLK, DH), jnp.float32)
    ones16 = jnp.ones((BLK, 16), jnp.float32)
    zeros16 = jnp.zeros((BLK, 16), jnp.float32)

    pdeg = _sc_deg(dst, ones16, zeros16)

    h1, g1l, g1r = _tc1(xp, W1, pdeg, pdeg)
    u1 = _sc_scatter(g1l, g1r, src, dst, zeros_h)
    h2, g2l, g2r = _tc2(u1, u1, u1, u1, h1, pdeg, pdeg,
                        b1.reshape(1, D), W2)
    u2 = _sc_scatter(g2l, g2r, src, dst, zeros_h)
    out = _tc3(u2, u2, u2, u2, h2, pdeg, pdeg, b2.reshape(1, D))
    return out[:N]


# speedup vs baseline: 3.3329x; 1.2676x over previous
import functools

import jax
import jax.numpy as jnp
from jax import lax
from jax.experimental import pallas as pl
from jax.experimental.pallas import tpu as pltpu
from jax.experimental.pallas import tpu_sc as plsc

N = 10000
E = 320000
D = 128

NW = 32
BLK = 128
K = 3
NSB = 27
NB = K * NSB
EPAD = NW * NB * BLK
NPAD = 10240
RPT = NPAD // 16

_mesh = plsc.VectorSubcoreMesh(core_axis_name="c", subcore_axis_name="s")


@functools.partial(
    pl.kernel,
    out_type=jax.ShapeDtypeStruct((2, NPAD, 16), jnp.float32),
    mesh=_mesh,
    compiler_params=pltpu.CompilerParams(use_tc_tiling_on_sc=False),
    scratch_types=[
        pltpu.VMEM((NB, BLK), jnp.int32),
        pltpu.VMEM((BLK, 16), jnp.float32),
        pltpu.VMEM((BLK, 16), jnp.float32),
        pltpu.VMEM_SHARED((NPAD, 16), jnp.float32),
    ],
)
def _sc_deg(dst_hbm, ones_hbm, zeros_hbm, out_hbm, idx_v, ones_v, zeros_v, acc):
    c = lax.axis_index("c")
    s = lax.axis_index("s")
    wid = s * 2 + c
    pltpu.sync_copy(dst_hbm.at[wid], idx_v)
    pltpu.sync_copy(ones_hbm, ones_v)
    pltpu.sync_copy(zeros_hbm, zeros_v)
    for b in range(RPT // BLK):
        pltpu.sync_copy(zeros_v, acc.at[pl.ds(s * RPT + b * BLK, BLK)])
    plsc.subcore_barrier()

    def body(j, carry):
        pltpu.sync_copy(ones_v, acc.at[idx_v.at[j]], add=True)
        return carry

    lax.fori_loop(0, NB, body, 0)
    plsc.subcore_barrier()
    pltpu.sync_copy(acc.at[pl.ds(s * RPT, RPT)], out_hbm.at[c, pl.ds(s * RPT, RPT)])


DH = D // 2


@functools.partial(
    pl.kernel,
    out_type=jax.ShapeDtypeStruct((2, 2, NPAD, DH), jnp.float32),
    mesh=_mesh,
    compiler_params=pltpu.CompilerParams(use_tc_tiling_on_sc=False),
    scratch_types=[
        pltpu.VMEM((NB, BLK), jnp.int32),
        pltpu.VMEM((NB, BLK), jnp.int32),
        pltpu.VMEM((2, K, BLK, DH), jnp.float32),
        pltpu.VMEM((BLK, DH), jnp.float32),
        pltpu.VMEM_SHARED((NPAD, DH), jnp.float32),
        pltpu.SemaphoreType.DMA,
        pltpu.SemaphoreType.DMA,
        pltpu.SemaphoreType.DMA,
        pltpu.SemaphoreType.DMA,
    ],
)
def _sc_scatter(gl_hbm, gr_hbm, src_hbm, dst_hbm, zeros_hbm, out_hbm,
                src_v, dst_v, bufs, zeros_v, acc, sem_ga, sem_gb,
                sem_sa, sem_sb):
    c = lax.axis_index("c")
    s = lax.axis_index("s")
    wid = s * 2 + c
    pltpu.sync_copy(src_hbm.at[wid], src_v)
    pltpu.sync_copy(dst_hbm.at[wid], dst_v)
    pltpu.sync_copy(zeros_hbm, zeros_v)
    gsem = (sem_ga, sem_gb)
    ssem = (sem_sa, sem_sb)
    for half in range(2):
        g_hbm = gl_hbm if half == 0 else gr_hbm

        def fire_g(sb, st):
            for t in range(K):
                pltpu.async_copy(g_hbm.at[src_v.at[sb * K + t]],
                                 bufs.at[st, t], gsem[st])

        def drain_g(sb, st):
            for t in range(K):
                pltpu.make_async_copy(g_hbm.at[src_v.at[sb * K + t]],
                                      bufs.at[st, t], gsem[st]).wait()

        def fire_s(sb, st):
            for t in range(K):
                pltpu.async_copy(bufs.at[st, t], acc.at[dst_v.at[sb * K + t]],
                                 ssem[st], add=True)

        def drain_s(sb, st):
            for t in range(K):
                pltpu.make_async_copy(bufs.at[st, t],
                                      acc.at[dst_v.at[sb * K + t]],
                                      ssem[st]).wait()

        for b in range(RPT // BLK):
            pltpu.sync_copy(zeros_v, acc.at[pl.ds(s * RPT + b * BLK, BLK)])
        plsc.subcore_barrier()

        fire_g(0, 0)

        def body(i, carry):
            sb0 = 2 * i
            sb1 = 2 * i + 1
            drain_g(sb0, 0)
            fire_g(sb1, 1)
            fire_s(sb0, 0)
            drain_s(sb0, 0)
            drain_g(sb1, 1)
            fire_g(sb1 + 1, 0)
            fire_s(sb1, 1)
            drain_s(sb1, 1)
            return carry

        lax.fori_loop(0, (NSB - 1) // 2, body, 0)
        drain_g(NSB - 1, 0)
        fire_s(NSB - 1, 0)
        drain_s(NSB - 1, 0)
        plsc.subcore_barrier()
        pltpu.sync_copy(acc.at[pl.ds(s * RPT, RPT)],
                        out_hbm.at[c, half, pl.ds(s * RPT, RPT)])


_TC_BLK = 1024
_TC_GRID = NPAD // _TC_BLK


def _rows_spec(width=D):
    return pl.BlockSpec((_TC_BLK, width), lambda i: (i, 0))


def _full_spec(shape):
    return pl.BlockSpec(shape, lambda i: (0, 0))


def _pdeg_spec(ci):
    return pl.BlockSpec((1, _TC_BLK, 16), lambda i, ci=ci: (ci, i, 0))


def _u_spec(ci, hi):
    return pl.BlockSpec((1, 1, _TC_BLK, DH), lambda i, ci=ci, hi=hi: (ci, hi, i, 0))


def _dinv(pa_ref, pb_ref):
    deg = 1.0 + pa_ref[0, :, :1] + pb_ref[0, :, :1]
    return lax.rsqrt(deg)


def _usum(u0l, u1l, u0r, u1r):
    return jnp.concatenate([u0l[0, 0] + u1l[0, 0], u0r[0, 0] + u1r[0, 0]],
                           axis=1)


def _tc1_body(x_ref, w_ref, pa_ref, pb_ref, h_ref, gl_ref, gr_ref):
    h = jnp.dot(x_ref[...], w_ref[...], preferred_element_type=jnp.float32)
    dinv = _dinv(pa_ref, pb_ref)
    h_ref[...] = h
    g = h * dinv
    gl_ref[...] = g[:, :DH]
    gr_ref[...] = g[:, DH:]


def _tc2_body(u0l, u1l, u0r, u1r, h1_ref, pa_ref, pb_ref, b1_ref, w2_ref,
              h2_ref, g2l_ref, g2r_ref):
    dinv = _dinv(pa_ref, pb_ref)
    a = dinv * _usum(u0l, u1l, u0r, u1r) + (dinv * dinv) * h1_ref[...] + b1_ref[...]
    o = jnp.maximum(a, 0.0)
    h2 = jnp.dot(o, w2_ref[...], preferred_element_type=jnp.float32)
    h2_ref[...] = h2
    g2 = h2 * dinv
    g2l_ref[...] = g2[:, :DH]
    g2r_ref[...] = g2[:, DH:]


def _tc3_body(u0l, u1l, u0r, u1r, h2_ref, pa_ref, pb_ref, b2_ref, out_ref):
    dinv = _dinv(pa_ref, pb_ref)
    out_ref[...] = (dinv * _usum(u0l, u1l, u0r, u1r)
                    + (dinv * dinv) * h2_ref[...] + b2_ref[...])


_half_out = [jax.ShapeDtypeStruct((NPAD, DH), jnp.float32)] * 2
_u_specs = [_u_spec(0, 0), _u_spec(1, 0), _u_spec(0, 1), _u_spec(1, 1)]

_tc1 = pl.pallas_call(
    _tc1_body,
    grid=(_TC_GRID,),
    in_specs=[_rows_spec(), _full_spec((D, D)), _pdeg_spec(0), _pdeg_spec(1)],
    out_specs=[_rows_spec(), _rows_spec(DH), _rows_spec(DH)],
    out_shape=[jax.ShapeDtypeStruct((NPAD, D), jnp.float32)] + _half_out,
)

_tc2 = pl.pallas_call(
    _tc2_body,
    grid=(_TC_GRID,),
    in_specs=_u_specs + [_rows_spec(), _pdeg_spec(0), _pdeg_spec(1),
                         _full_spec((1, D)), _full_spec((D, D))],
    out_specs=[_rows_spec(), _rows_spec(DH), _rows_spec(DH)],
    out_shape=[jax.ShapeDtypeStruct((NPAD, D), jnp.float32)] + _half_out,
)

_tc3 = pl.pallas_call(
    _tc3_body,
    grid=(_TC_GRID,),
    in_specs=_u_specs + [_rows_spec(), _pdeg_spec(0), _pdeg_spec(1),
                         _full_spec((1, D))],
    out_specs=_rows_spec(),
    out_shape=jax.ShapeDtypeStruct((NPAD, D), jnp.float32),
)


def kernel(x, edge_index, W1, b1, W2, b2):
    ei = edge_index.astype(jnp.int32)
    pad = N + jnp.arange(EPAD - E, dtype=jnp.int32) % (NPAD - N)
    src = jnp.concatenate([ei[0], pad]).reshape(NW, NB, BLK)
    dst = jnp.concatenate([ei[1], pad]).reshape(NW, NB, BLK)
    xp = jnp.pad(x, ((0, NPAD - N), (0, 0)))
    zeros_h = jnp.zeros((BLK, DH), jnp.float32)
    ones16 = jnp.ones((BLK, 16), jnp.float32)
    zeros16 = jnp.zeros((BLK, 16), jnp.float32)

    pdeg = _sc_deg(dst, ones16, zeros16)

    h1, g1l, g1r = _tc1(xp, W1, pdeg, pdeg)
    u1 = _sc_scatter(g1l, g1r, src, dst, zeros_h)
    h2, g2l, g2r = _tc2(u1, u1, u1, u1, h1, pdeg, pdeg,
                        b1.reshape(1, D), W2)
    u2 = _sc_scatter(g2l, g2r, src, dst, zeros_h)
    out = _tc3(u2, u2, u2, u2, h2, pdeg, pdeg, b2.reshape(1, D))
    return out[:N]
